# baseline (speedup 1.0000x reference)
import functools

import jax
import jax.numpy as jnp
from jax import lax
from jax.experimental import pallas as pl
from jax.experimental.pallas import tpu as pltpu
from jax.experimental.pallas import tpu_sc as plsc

EPS = 0.01
TOPK = 8
NEG = -1e30


def _ssq_kernel(v_ref, ssq_ref):
    r = pl.program_id(1)
    v = v_ref[0]
    part = jnp.sum(v * v)

    @pl.when(r == 0)
    def _init():
        ssq_ref[...] = jnp.reshape(part, (1, 1, 1))

    @pl.when(r != 0)
    def _acc():
        ssq_ref[...] += jnp.reshape(part, (1, 1, 1))


def _tokens_kernel(x_ref, v_ref, w_ref, c_ref, t_ref, p_ref):
    x = x_ref[0]
    v = v_ref[0]
    w = w_ref[...]
    c = c_ref[...][0]
    a = jnp.dot(x, w, preferred_element_type=jnp.float32)
    xp = x + c * v
    tp = jnp.dot(xp, w, preferred_element_type=jnp.float32)
    t_ref[0] = a / (jnp.sqrt(jnp.sum(a * a, axis=1, keepdims=True)) + 1e-6)
    p_ref[0] = tp / (jnp.sqrt(jnp.sum(tp * tp, axis=1, keepdims=True)) + 1e-6)


def _sims_kernel(t_ref, p_ref, key_ref, val_ref, *, rb, n):
    r = pl.program_id(1)

    rows_t = t_ref[0, pl.ds(r * rb, rb), :]
    rows_p = p_ref[0, pl.ds(r * rb, rb), :]
    s_t = jnp.dot(rows_t, t_ref[0].T, preferred_element_type=jnp.float32)
    s_p = jnp.dot(rows_p, p_ref[0].T, preferred_element_type=jnp.float32)

    col = jax.lax.broadcasted_iota(jnp.int32, (rb, n), 1)
    rowg = jax.lax.broadcasted_iota(jnp.int32, (rb, n), 0) + r * rb
    key_ref[...] = jnp.where(col == rowg, NEG, s_t)
    d = s_p - s_t
    val_ref[...] = d * d


def _make_sc_topk(total_rows, n, rows_blk):
    info = plsc.get_sparse_core_info()
    nc, ns, lanes = info.num_cores, info.num_subcores, info.num_lanes
    nw = nc * ns
    rows_per_w = total_rows // nw
    n_blocks = rows_per_w // rows_blk
    n_chunks = n // lanes
    mesh = plsc.VectorSubcoreMesh(core_axis_name="c", subcore_axis_name="s")

    @functools.partial(
        pl.kernel,
        mesh=mesh,
        compiler_params=pltpu.CompilerParams(needs_layout_passes=False),
        out_type=jax.ShapeDtypeStruct((nw, lanes), jnp.float32),
        scratch_types=[
            pltpu.VMEM((rows_blk, n), jnp.float32),
            pltpu.VMEM((rows_blk, n), jnp.float32),
            pltpu.VMEM((lanes,), jnp.float32),
            pltpu.VMEM((lanes,), jnp.float32),
            pltpu.VMEM((lanes,), jnp.float32),
        ],
    )
    def sc_topk(key_hbm, val_hbm, out_hbm, k_buf, v_buf, acc_ref, tk_ref, tv_ref):
        wid = lax.axis_index("s") * nc + lax.axis_index("c")
        base = wid * rows_per_w
        lane_iota = lax.iota(jnp.int32, lanes)
        topmask = lane_iota < TOPK
        zeros = jnp.zeros((lanes,), jnp.float32)
        acc_ref[...] = zeros

        def block_body(blk, _):
            row0 = base + blk * rows_blk
            pltpu.sync_copy(key_hbm.at[pl.ds(row0, rows_blk), :], k_buf)
            pltpu.sync_copy(val_hbm.at[pl.ds(row0, rows_blk), :], v_buf)

            for rr in range(rows_blk):
                tk_ref[...] = jnp.full((lanes,), NEG, jnp.float32)
                tv_ref[...] = zeros

                def chunk_body(i, carry):
                    ck = k_buf[rr, pl.ds(i * lanes, lanes)]
                    cv = v_buf[rr, pl.ds(i * lanes, lanes)]
                    ck, cv = plsc.sort_key_val(ck, cv)
                    tk = tk_ref[...]
                    tv = tv_ref[...]
                    m = tk >= ck
                    nk = jnp.where(m, tk, ck)
                    nv = jnp.where(m, tv, cv)
                    sk, sv = plsc.sort_key_val(nk, nv, descending=True)
                    tk_ref[...] = sk
                    tv_ref[...] = sv
                    return 0

                lax.fori_loop(0, n_chunks, chunk_body, 0)
                acc_ref[...] = acc_ref[...] + jnp.where(topmask, tv_ref[...], 0.0)
            return 0

        lax.fori_loop(0, n_blocks, block_body, 0)
        pltpu.sync_copy(acc_ref, out_hbm.at[wid])

    return sc_topk


def kernel(x_t, v_pred, W):
    bsz, n, d = x_t.shape
    h = W.shape[1]
    ra = 512
    rb = 256

    ssq = pl.pallas_call(
        _ssq_kernel,
        grid=(bsz, n // ra),
        in_specs=[pl.BlockSpec((1, ra, d), lambda b, r: (b, r, 0))],
        out_specs=pl.BlockSpec((1, 1, 1), lambda b, r: (b, 0, 0)),
        out_shape=jax.ShapeDtypeStruct((bsz, 1, 1), jnp.float32),
    )(v_pred)

    rms = jnp.sqrt(ssq / (n * d) + 1e-6)
    c = EPS / rms

    t_hat, p_hat = pl.pallas_call(
        _tokens_kernel,
        grid=(bsz, n // ra),
        in_specs=[
            pl.BlockSpec((1, ra, d), lambda b, r: (b, r, 0)),
            pl.BlockSpec((1, ra, d), lambda b, r: (b, r, 0)),
            pl.BlockSpec((d, h), lambda b, r: (0, 0)),
            pl.BlockSpec((1, 1, 1), lambda b, r: (b, 0, 0)),
        ],
        out_specs=[
            pl.BlockSpec((1, ra, h), lambda b, r: (b, r, 0)),
            pl.BlockSpec((1, ra, h), lambda b, r: (b, r, 0)),
        ],
        out_shape=[
            jax.ShapeDtypeStruct((bsz, n, h), jnp.float32),
            jax.ShapeDtypeStruct((bsz, n, h), jnp.float32),
        ],
    )(x_t, v_pred, W, c)

    nblk = n // rb
    keys, vals = pl.pallas_call(
        functools.partial(_sims_kernel, rb=rb, n=n),
        grid=(bsz, nblk),
        in_specs=[
            pl.BlockSpec((1, n, h), lambda b, r: (b, 0, 0)),
            pl.BlockSpec((1, n, h), lambda b, r: (b, 0, 0)),
        ],
        out_specs=[
            pl.BlockSpec((rb, n), lambda b, r: (b * nblk + r, 0)),
            pl.BlockSpec((rb, n), lambda b, r: (b * nblk + r, 0)),
        ],
        out_shape=[
            jax.ShapeDtypeStruct((bsz * n, n), jnp.float32),
            jax.ShapeDtypeStruct((bsz * n, n), jnp.float32),
        ],
    )(t_hat, p_hat)

    sc_fn = _make_sc_topk(bsz * n, n, rows_blk=8)
    partial = sc_fn(keys, vals)

    return jnp.sum(partial) / bsz


# speedup vs baseline: 5.7480x; 5.7480x over previous
import functools

import jax
import jax.numpy as jnp
from jax.experimental import pallas as pl
from jax.experimental.pallas import tpu as pltpu

EPS = 0.01
TOPK = 8


def _fused_kernel(x_ref, v_ref, w_ref, out_ref, t_scr, p_scr, ssq_s,
                  *, ra, rb, n, d, nra, nrb):
    b = pl.program_id(0)
    j = pl.program_id(1)

    @pl.when(j < nra)
    def _ssq():
        v = v_ref[0]
        part = jnp.sum(v * v)

        @pl.when(j == 0)
        def _init():
            ssq_s[0] = part

        @pl.when(j != 0)
        def _acc():
            ssq_s[0] += part

    @pl.when((j >= nra) & (j < 2 * nra))
    def _tokens():
        x = x_ref[0]
        v = v_ref[0]
        w = w_ref[...]
        rms = jnp.sqrt(ssq_s[0] / (n * d) + 1e-6)
        c = EPS / rms
        a = jnp.dot(x, w, preferred_element_type=jnp.float32)
        xp = x + c * v
        tp = jnp.dot(xp, w, preferred_element_type=jnp.float32)
        sl = pl.ds((j - nra) * ra, ra)
        t_scr[sl, :] = a / (jnp.sqrt(jnp.sum(a * a, axis=1, keepdims=True)) + 1e-6)
        p_scr[sl, :] = tp / (jnp.sqrt(jnp.sum(tp * tp, axis=1, keepdims=True)) + 1e-6)

    @pl.when(j >= 2 * nra)
    def _loss():
        r = j - 2 * nra
        rows_t = t_scr[pl.ds(r * rb, rb), :]
        rows_p = p_scr[pl.ds(r * rb, rb), :]
        s_t = jnp.dot(rows_t, t_scr[...].T, preferred_element_type=jnp.float32)
        s_p = jnp.dot(rows_p, p_scr[...].T, preferred_element_type=jnp.float32)

        col = jax.lax.broadcasted_iota(jnp.int32, (rb, n), 1)
        rowg = jax.lax.broadcasted_iota(jnp.int32, (rb, n), 0) + r * rb
        neg = jnp.float32(-jnp.inf)
        work = jnp.where(col == rowg, neg, s_t)
        dd = s_p - s_t
        d2 = dd * dd

        acc = jnp.float32(0.0)
        for _ in range(TOPK):
            m = jnp.max(work, axis=1, keepdims=True)
            sel = work == m
            acc += jnp.sum(jnp.where(sel, d2, 0.0))
            work = jnp.where(sel, neg, work)

        @pl.when((b == 0) & (r == 0))
        def _out_init():
            out_ref[...] = jnp.reshape(acc, (1, 1))

        @pl.when((b != 0) | (r != 0))
        def _out_acc():
            out_ref[...] += jnp.reshape(acc, (1, 1))


def kernel(x_t, v_pred, W):
    bsz, n, d = x_t.shape
    h = W.shape[1]
    ra = 512
    rb = 256
    nra = n // ra
    nrb = n // rb

    def x_idx(b, j):
        return (b, jnp.clip(j - nra, 0, nra - 1), 0)

    def v_idx(b, j):
        return (b, jnp.where(j < 2 * nra, jnp.remainder(j, nra), nra - 1), 0)

    out = pl.pallas_call(
        functools.partial(_fused_kernel, ra=ra, rb=rb, n=n, d=d,
                          nra=nra, nrb=nrb),
        grid=(bsz, 2 * nra + nrb),
        in_specs=[
            pl.BlockSpec((1, ra, d), x_idx),
            pl.BlockSpec((1, ra, d), v_idx),
            pl.BlockSpec((d, h), lambda b, j: (0, 0)),
        ],
        out_specs=pl.BlockSpec((1, 1), lambda b, j: (0, 0)),
        out_shape=jax.ShapeDtypeStruct((1, 1), jnp.float32),
        scratch_shapes=[
            pltpu.VMEM((n, h), jnp.float32),
            pltpu.VMEM((n, h), jnp.float32),
            pltpu.SMEM((1,), jnp.float32),
        ],
    )(x_t, v_pred, W)

    return out[0, 0] / bsz


# speedup vs baseline: 7.1231x; 1.2392x over previous
import functools

import jax
import jax.numpy as jnp
from jax.experimental import pallas as pl
from jax.experimental.pallas import tpu as pltpu

EPS = 0.01
TOPK = 8


def _fused_kernel(x_ref, v_ref, w_ref, out_ref, t_scr, p_scr, ssq_s,
                  *, ra, rb, n, d, nra, nrb):
    b = pl.program_id(0)
    j = pl.program_id(1)

    @pl.when(j < nra)
    def _ssq():
        v = v_ref[0]
        part = jnp.sum(v * v)

        @pl.when(j == 0)
        def _init():
            ssq_s[0] = part

        @pl.when(j != 0)
        def _acc():
            ssq_s[0] += part

    @pl.when((j >= nra) & (j < 2 * nra))
    def _tokens():
        x = x_ref[0]
        v = v_ref[0]
        w = w_ref[...]
        rms = jnp.sqrt(ssq_s[0] / (n * d) + 1e-6)
        c = EPS / rms
        a = jnp.dot(x, w, preferred_element_type=jnp.float32)
        xp = x + c * v
        tp = jnp.dot(xp, w, preferred_element_type=jnp.float32)
        sl = pl.ds((j - nra) * ra, ra)
        t_scr[sl, :] = a / (jnp.sqrt(jnp.sum(a * a, axis=1, keepdims=True)) + 1e-6)
        p_scr[sl, :] = tp / (jnp.sqrt(jnp.sum(tp * tp, axis=1, keepdims=True)) + 1e-6)

    @pl.when(j >= 2 * nra)
    def _loss():
        r = j - 2 * nra
        rows_t = t_scr[pl.ds(r * rb, rb), :]
        rows_p = p_scr[pl.ds(r * rb, rb), :]
        s_t = jnp.dot(rows_t, t_scr[...].T, preferred_element_type=jnp.float32)
        s_p = jnp.dot(rows_p, p_scr[...].T, preferred_element_type=jnp.float32)

        col = jax.lax.broadcasted_iota(jnp.int32, (rb, n), 1)
        rowg = jax.lax.broadcasted_iota(jnp.int32, (rb, n), 0) + r * rb
        neg = jnp.float32(-jnp.inf)
        w0 = jnp.where(col == rowg, neg, s_t)

        half = n // 2
        a = w0[:, :half]
        bb = w0[:, half:]
        hi = jnp.maximum(a, bb)
        lo = jnp.minimum(a, bb)
        for _ in range(TOPK - 1):
            m = jnp.max(hi, axis=1, keepdims=True)
            sel = hi == m
            hi = jnp.where(sel, lo, hi)
            lo = jnp.where(sel, neg, lo)
        t8 = jnp.max(hi, axis=1, keepdims=True)

        dd = s_p - s_t
        acc = jnp.sum(jnp.where(w0 >= t8, dd * dd, 0.0))

        @pl.when((b == 0) & (r == 0))
        def _out_init():
            out_ref[...] = jnp.reshape(acc, (1, 1))

        @pl.when((b != 0) | (r != 0))
        def _out_acc():
            out_ref[...] += jnp.reshape(acc, (1, 1))


def kernel(x_t, v_pred, W):
    bsz, n, d = x_t.shape
    h = W.shape[1]
    ra = 512
    rb = 256
    nra = n // ra
    nrb = n // rb

    def x_idx(b, j):
        return (b, jnp.clip(j - nra, 0, nra - 1), 0)

    def v_idx(b, j):
        return (b, jnp.where(j < 2 * nra, jnp.remainder(j, nra), nra - 1), 0)

    out = pl.pallas_call(
        functools.partial(_fused_kernel, ra=ra, rb=rb, n=n, d=d,
                          nra=nra, nrb=nrb),
        grid=(bsz, 2 * nra + nrb),
        in_specs=[
            pl.BlockSpec((1, ra, d), x_idx),
            pl.BlockSpec((1, ra, d), v_idx),
            pl.BlockSpec((d, h), lambda b, j: (0, 0)),
        ],
        out_specs=pl.BlockSpec((1, 1), lambda b, j: (0, 0)),
        out_shape=jax.ShapeDtypeStruct((1, 1), jnp.float32),
        scratch_shapes=[
            pltpu.VMEM((n, h), jnp.float32),
            pltpu.VMEM((n, h), jnp.float32),
            pltpu.SMEM((1,), jnp.float32),
        ],
    )(x_t, v_pred, W)

    return out[0, 0] / bsz
